# Initial kernel scaffold; baseline (speedup 1.0000x reference)
import jax
import jax.numpy as jnp
from jax.experimental import pallas as pl


def kernel(x, edge_index, W1, V1, b1, W2, V2, b2):
    raise NotImplementedError("write your pallas kernel here")


# speedup vs baseline: 10.2899x; 10.2899x over previous
import functools

import jax
import jax.numpy as jnp
from jax import lax
from jax.experimental import pallas as pl
from jax.experimental.pallas import tpu as pltpu
from jax.experimental.pallas import tpu_sc as plsc

N = 10000
D = 128
NC, NS = 2, 16
NW = NC * NS
CHUNK = 128
N_SP = 10112
RPT = N_SP // NS

_mesh = plsc.VectorSubcoreMesh(core_axis_name="c", subcore_axis_name="s")


N_DEG = 10240
DEG_PT = N_DEG // NS


def _sc_deg(chunks):
    @functools.partial(
        pl.kernel,
        out_type=jax.ShapeDtypeStruct((NC, N_DEG), jnp.float32),
        mesh=_mesh,
        compiler_params=pltpu.CompilerParams(needs_layout_passes=False),
        scratch_types=[
            pltpu.VMEM((chunks * CHUNK,), jnp.int32),
            pltpu.VMEM((NS * DEG_PT,), jnp.float32),
            pltpu.VMEM_SHARED((NS, N_DEG), jnp.float32),
        ],
    )
    def deg_kernel(col_hbm, out_hbm, col_v, deg_loc, stage_sh):
        c = lax.axis_index("c")
        s = lax.axis_index("s")
        wid = c * NS + s
        pltpu.sync_copy(col_hbm.at[wid], col_v)

        def zbody(i, carry):
            deg_loc[pl.ds(i * 16, 16)] = jnp.zeros((16,), jnp.float32)
            return carry

        lax.fori_loop(0, N_DEG // 16, zbody, 0)
        lanes = lax.iota(jnp.int32, 16)

        def body(j, carry):
            cols = col_v[pl.ds(j * 16, 16)]
            for k in range(16):
                mk = lanes == k
                v = plsc.load_gather(deg_loc, [cols], mask=mk)
                plsc.store_scatter(deg_loc, [cols], v + 1.0, mask=mk)
            return carry

        lax.fori_loop(0, chunks * CHUNK // 16, body, 0)
        pltpu.sync_copy(deg_loc.at[pl.ds(0, N_DEG)], stage_sh.at[s])
        plsc.subcore_barrier()
        span = s * DEG_PT
        for p in range(NS):
            pltpu.sync_copy(stage_sh.at[p, pl.ds(span, DEG_PT)],
                            deg_loc.at[pl.ds(p * DEG_PT, DEG_PT)])

        def rbody(i, carry):
            acc = deg_loc[pl.ds(i * 16, 16)]
            for p in range(1, NS):
                acc = acc + deg_loc[pl.ds(p * DEG_PT + i * 16, 16)]
            deg_loc[pl.ds(i * 16, 16)] = acc
            return carry

        lax.fori_loop(0, DEG_PT // 16, rbody, 0)
        pltpu.sync_copy(deg_loc.at[pl.ds(0, DEG_PT)],
                        out_hbm.at[c, pl.ds(span, DEG_PT)])

    return deg_kernel


def _sc_agg(chunks):
    @functools.partial(
        pl.kernel,
        out_type=jax.ShapeDtypeStruct((NC, N_SP, D), jnp.float32),
        mesh=_mesh,
        scratch_types=[
            pltpu.VMEM((chunks, CHUNK), jnp.int32),
            pltpu.VMEM((chunks, CHUNK), jnp.int32),
            pltpu.VMEM((CHUNK, D), jnp.float32),
            pltpu.VMEM_SHARED((N_SP, D), jnp.float32),
            pltpu.SemaphoreType.DMA,
        ],
    )
    def agg_kernel(row_hbm, col_hbm, h_hbm, zeros_hbm, out_hbm,
                   row_v, col_v, buf, agg_sh, sem):
        c = lax.axis_index("c")
        s = lax.axis_index("s")
        wid = c * NS + s
        base = s * RPT
        pltpu.sync_copy(zeros_hbm.at[pl.ds(base, RPT)], agg_sh.at[pl.ds(base, RPT)])
        pltpu.sync_copy(row_hbm.at[wid], row_v)
        pltpu.sync_copy(col_hbm.at[wid], col_v)
        plsc.subcore_barrier()

        def body(j, carry):
            pltpu.async_copy(h_hbm.at[row_v.at[j]], buf, sem).wait()
            pltpu.sync_copy(buf, agg_sh.at[col_v.at[j]], add=True)
            return carry

        lax.fori_loop(0, chunks, body, 0)
        plsc.subcore_barrier()
        pltpu.sync_copy(agg_sh.at[pl.ds(base, RPT)], out_hbm.at[c, pl.ds(base, RPT)])

    return agg_kernel


def _tc_first(degp, x, W1, V1):
    bk = 2000
    grid = N // bk

    def body(degp_ref, x_ref, w_ref, v_ref, dinv_ref, h1p_ref, xv1_ref):
        deg = degp_ref[0, :, :] + degp_ref[1, :, :]
        dinv = jnp.where(deg > 0, lax.rsqrt(deg), 0.0)
        xb = x_ref[...]
        h = jnp.dot(xb, w_ref[...], preferred_element_type=jnp.float32)
        h1p_ref[...] = dinv * h
        xv1_ref[...] = jnp.dot(xb, v_ref[...], preferred_element_type=jnp.float32)
        dinv_ref[...] = jnp.broadcast_to(dinv, (bk, 8))

    return pl.pallas_call(
        body,
        grid=(grid,),
        in_specs=[
            pl.BlockSpec((2, bk, 1), lambda i: (0, i, 0)),
            pl.BlockSpec((bk, D), lambda i: (i, 0)),
            pl.BlockSpec((D, D), lambda i: (0, 0)),
            pl.BlockSpec((D, D), lambda i: (0, 0)),
        ],
        out_specs=[
            pl.BlockSpec((bk, 8), lambda i: (i, 0)),
            pl.BlockSpec((bk, D), lambda i: (i, 0)),
            pl.BlockSpec((bk, D), lambda i: (i, 0)),
        ],
        out_shape=[
            jax.ShapeDtypeStruct((N, 8), jnp.float32),
            jax.ShapeDtypeStruct((N, D), jnp.float32),
            jax.ShapeDtypeStruct((N, D), jnp.float32),
        ],
    )(degp, x, W1, V1)


def _tc_mid(aggp, dinv8, xv1, b1, W2, V2):
    bk = 2000
    grid = N // bk

    def body(aggp_ref, dinv_ref, xv_ref, b_ref, w_ref, v_ref, h2p_ref, xv2_ref):
        dinv = dinv_ref[:, 0:1]
        agg = aggp_ref[0, :, :] + aggp_ref[1, :, :]
        h = jnp.maximum(dinv * agg + xv_ref[...] + b_ref[...], 0.0)
        hw = jnp.dot(h, w_ref[...], preferred_element_type=jnp.float32)
        h2p_ref[...] = dinv * hw
        xv2_ref[...] = jnp.dot(h, v_ref[...], preferred_element_type=jnp.float32)

    return pl.pallas_call(
        body,
        grid=(grid,),
        in_specs=[
            pl.BlockSpec((2, bk, D), lambda i: (0, i, 0)),
            pl.BlockSpec((bk, 8), lambda i: (i, 0)),
            pl.BlockSpec((bk, D), lambda i: (i, 0)),
            pl.BlockSpec((1, D), lambda i: (0, 0)),
            pl.BlockSpec((D, D), lambda i: (0, 0)),
            pl.BlockSpec((D, D), lambda i: (0, 0)),
        ],
        out_specs=[
            pl.BlockSpec((bk, D), lambda i: (i, 0)),
            pl.BlockSpec((bk, D), lambda i: (i, 0)),
        ],
        out_shape=[
            jax.ShapeDtypeStruct((N, D), jnp.float32),
            jax.ShapeDtypeStruct((N, D), jnp.float32),
        ],
    )(aggp, dinv8, xv1, b1, W2, V2)


def _tc_last(aggp, dinv8, xv2, b2):
    bk = 2000
    grid = N // bk

    def body(aggp_ref, dinv_ref, xv_ref, b_ref, out_ref):
        dinv = dinv_ref[:, 0:1]
        agg = aggp_ref[0, :, :] + aggp_ref[1, :, :]
        out_ref[...] = jnp.maximum(dinv * agg + xv_ref[...] + b_ref[...], 0.0)

    return pl.pallas_call(
        body,
        grid=(grid,),
        in_specs=[
            pl.BlockSpec((2, bk, D), lambda i: (0, i, 0)),
            pl.BlockSpec((bk, 8), lambda i: (i, 0)),
            pl.BlockSpec((bk, D), lambda i: (i, 0)),
            pl.BlockSpec((1, D), lambda i: (0, 0)),
        ],
        out_specs=pl.BlockSpec((bk, D), lambda i: (i, 0)),
        out_shape=jax.ShapeDtypeStruct((N, D), jnp.float32),
    )(aggp, dinv8, xv2, b2)


def kernel(x, edge_index, W1, V1, b1, W2, V2, b2):
    e = edge_index.shape[1]
    ept = NW * CHUNK
    chunks = -(-e // ept)
    e_pad = chunks * ept
    ei = edge_index.astype(jnp.int32)
    row = jnp.concatenate([ei[0], jnp.zeros((e_pad - e,), jnp.int32)])
    col = jnp.concatenate([ei[1], jnp.full((e_pad - e,), N, jnp.int32)])
    row_r = row.reshape(NW, chunks, CHUNK)
    col_r = col.reshape(NW, chunks, CHUNK)

    zerosd = jnp.zeros((N_SP, D), jnp.float32)
    b1r = b1.reshape(1, D)
    b2r = b2.reshape(1, D)

    col_f = col.reshape(NW, chunks * CHUNK)
    degp = _sc_deg(chunks)(col_f).reshape(NC, N_DEG, 1)
    dinv8, h1p, xv1 = _tc_first(degp, x, W1, V1)
    agg1 = _sc_agg(chunks)(row_r, col_r, h1p, zerosd)
    h2p, xv2 = _tc_mid(agg1, dinv8, xv1, b1r, W2, V2)
    agg2 = _sc_agg(chunks)(row_r, col_r, h2p, zerosd)
    return _tc_last(agg2, dinv8, xv2, b2r)
